# baseline (speedup 1.0000x reference)
import functools

import jax
import jax.numpy as jnp
from jax import lax
from jax.experimental import pallas as pl
from jax.experimental.pallas import tpu as pltpu
from jax.experimental.pallas import tpu_sc as plsc


def _pack_bf16_halves(y):
    D = y.shape[1]
    a = lax.bitcast_convert_type(y[:, : D // 2], jnp.uint32)
    b = lax.bitcast_convert_type(y[:, D // 2:], jnp.uint32)
    ar = (a + 0x7FFF + ((a >> 16) & 1)) >> 16
    br = (b + 0x7FFF + ((b >> 16) & 1)) >> 16
    return lax.bitcast_convert_type((br << 16) | ar, jnp.int32)


def _proj_body(x_ref, w1a_ref, w1b_ref, b1_ref, y1_ref, y2_ref):
    xb = x_ref[...]
    y1 = (
        jnp.dot(xb, w1a_ref[...], preferred_element_type=jnp.float32)
        + b1_ref[...]
    )
    y2 = jnp.dot(xb, w1b_ref[...], preferred_element_type=jnp.float32)
    y1_ref[...] = _pack_bf16_halves(y1)
    y2_ref[...] = _pack_bf16_halves(y2)


def _node_proj(x, W1a, W1b, b1):
    N, D = x.shape
    NB = 1000
    return pl.pallas_call(
        _proj_body,
        grid=(N // NB,),
        in_specs=[
            pl.BlockSpec((NB, D), lambda i: (i, 0)),
            pl.BlockSpec((D, D), lambda i: (0, 0)),
            pl.BlockSpec((D, D), lambda i: (0, 0)),
            pl.BlockSpec((1, D), lambda i: (0, 0)),
        ],
        out_specs=[
            pl.BlockSpec((NB, D // 2), lambda i: (i, 0)),
            pl.BlockSpec((NB, D // 2), lambda i: (i, 0)),
        ],
        out_shape=[jax.ShapeDtypeStruct((N, D // 2), jnp.int32)] * 2,
    )(x, W1a, W1b, b1.reshape(1, D))


_NC = 2
_NS = 16
_NW = _NC * _NS
_K = 64
_NB = 4


def _sc_gather(y1i, y2i, src2, dst2, e_pad):
    W = y1i.shape[1]
    nch = e_pad // (_NW * _K)
    pw = nch * _K
    mesh = plsc.VectorSubcoreMesh(core_axis_name="c", subcore_axis_name="s")

    @functools.partial(
        pl.kernel,
        mesh=mesh,
        out_type=[jax.ShapeDtypeStruct((e_pad, W), jnp.int32)] * 2,
        scratch_types=[
            pltpu.VMEM((nch, _K), jnp.int32),
            pltpu.VMEM((nch, _K), jnp.int32),
            [pltpu.VMEM((_K, W), jnp.int32)] * _NB,
            [pltpu.VMEM((_K, W), jnp.int32)] * _NB,
            [pltpu.SemaphoreType.DMA] * _NB,
            [pltpu.SemaphoreType.DMA] * _NB,
        ],
    )
    def gather_kernel(y1_hbm, y2_hbm, src_hbm, dst_hbm, g1_hbm, g2_hbm,
                      isa, ida, r1, r2, sg, sw):
        wid = lax.axis_index("s") * _NC + lax.axis_index("c")
        base_row = wid * nch
        base = wid * pw

        pltpu.sync_copy(src_hbm.at[pl.ds(base_row, nch)], isa)
        pltpu.sync_copy(dst_hbm.at[pl.ds(base_row, nch)], ida)

        def issue_gathers(j, b):
            pltpu.async_copy(y1_hbm.at[isa.at[j]], r1[b], sg[b])
            pltpu.async_copy(y2_hbm.at[ida.at[j]], r2[b], sg[b])

        def wait_gathers(j, b):
            pltpu.make_async_copy(y1_hbm.at[isa.at[j]], r1[b], sg[b]).wait()
            pltpu.make_async_copy(y2_hbm.at[ida.at[j]], r2[b], sg[b]).wait()

        def o1(j):
            return g1_hbm.at[pl.ds(base + j * _K, _K)]

        def o2(j):
            return g2_hbm.at[pl.ds(base + j * _K, _K)]

        def issue_writes(j, b):
            pltpu.async_copy(r1[b], o1(j), sw[b])
            pltpu.async_copy(r2[b], o2(j), sw[b])

        def wait_writes(j, b):
            pltpu.make_async_copy(r1[b], o1(j), sw[b]).wait()
            pltpu.make_async_copy(r2[b], o2(j), sw[b]).wait()

        def sub(j, b, wait_w, issue_n):
            wait_gathers(j, b)
            issue_writes(j, b)
            if issue_n:
                bn = (b + 2) % _NB
                if wait_w:
                    wait_writes(j - 2, bn)
                issue_gathers(j + 2, bn)

        issue_gathers(0, 0)
        issue_gathers(1, 1)
        sub(0, 0, wait_w=False, issue_n=True)
        sub(1, 1, wait_w=False, issue_n=True)

        def step(q, carry):
            c0 = 4 * q + 2
            sub(c0, 2, wait_w=True, issue_n=True)
            sub(c0 + 1, 3, wait_w=True, issue_n=True)
            sub(c0 + 2, 0, wait_w=True, issue_n=True)
            sub(c0 + 3, 1, wait_w=True, issue_n=True)
            return carry

        lax.fori_loop(0, (nch - 4) // 4, step, 0)

        sub(nch - 2, 2, wait_w=False, issue_n=False)
        sub(nch - 1, 3, wait_w=False, issue_n=False)
        wait_writes(nch - 4, 0)
        wait_writes(nch - 3, 1)
        wait_writes(nch - 2, 2)
        wait_writes(nch - 1, 3)

    return gather_kernel(y1i, y2i, src2, dst2)


def _unpack_lo(u):
    return lax.bitcast_convert_type(u << 16, jnp.float32)


def _unpack_hi(u):
    return lax.bitcast_convert_type(u & jnp.uint32(0xFFFF0000), jnp.float32)


def _edge_body(g1_ref, g2_ref, ea_ref, w2a_ref, w2b_ref, b2_ref, gm_ref,
               bt_ref, o_ref):
    g1 = lax.bitcast_convert_type(g1_ref[...], jnp.uint32)
    g2 = lax.bitcast_convert_type(g2_ref[...], jnp.uint32)
    ta = _unpack_lo(g1) + _unpack_lo(g2)
    tb = _unpack_hi(g1) + _unpack_hi(g2)
    ha = jnp.where(ta > 0, ta, jnp.exp(ta) - 1.0)
    hb = jnp.where(tb > 0, tb, jnp.exp(tb) - 1.0)
    score = (
        jnp.dot(ha, w2a_ref[...], preferred_element_type=jnp.float32)
        + jnp.dot(hb, w2b_ref[...], preferred_element_type=jnp.float32)
        + b2_ref[...]
    )
    r = ea_ref[...] + score
    mu = jnp.mean(r, axis=1, keepdims=True)
    c = r - mu
    var = jnp.mean(c * c, axis=1, keepdims=True)
    o_ref[...] = c * lax.rsqrt(var + 1e-5) * gm_ref[...] + bt_ref[...]


def _edge_stage(g1, g2, edge_attr, W2, b2, gamma, beta, E):
    D = edge_attr.shape[1]
    EB = 1600
    return pl.pallas_call(
        _edge_body,
        grid=(E // EB,),
        in_specs=[
            pl.BlockSpec((EB, D // 2), lambda i: (i, 0)),
            pl.BlockSpec((EB, D // 2), lambda i: (i, 0)),
            pl.BlockSpec((EB, D), lambda i: (i, 0)),
            pl.BlockSpec((D // 2, D), lambda i: (0, 0)),
            pl.BlockSpec((D // 2, D), lambda i: (0, 0)),
            pl.BlockSpec((1, D), lambda i: (0, 0)),
            pl.BlockSpec((1, D), lambda i: (0, 0)),
            pl.BlockSpec((1, D), lambda i: (0, 0)),
        ],
        out_specs=pl.BlockSpec((EB, D), lambda i: (i, 0)),
        out_shape=jax.ShapeDtypeStruct((E, D), jnp.float32),
    )(g1, g2, edge_attr, W2[: D // 2], W2[D // 2:], b2.reshape(1, D),
      gamma.reshape(1, D), beta.reshape(1, D))


def kernel(x, edge_index, edge_attr, W1, b1, W2, b2, gamma, beta):
    N, D = x.shape
    E = edge_index.shape[1]

    W1a = W1[:D]
    W1b = W1[D:]
    y1i, y2i = _node_proj(x, W1a, W1b, b1)

    half = E // 2
    quantum = _NB * _NW * _K
    h_pad = ((half + quantum - 1) // quantum) * quantum
    src = edge_index[0].astype(jnp.int32)
    dst = edge_index[1].astype(jnp.int32)
    outs = []
    for lo, n in ((0, half), (half, E - half)):
        s = lax.dynamic_slice_in_dim(src, lo, n)
        d = lax.dynamic_slice_in_dim(dst, lo, n)
        if h_pad != n:
            pad = h_pad - n
            s = jnp.concatenate([s, jnp.zeros((pad,), jnp.int32)])
            d = jnp.concatenate([d, jnp.zeros((pad,), jnp.int32)])
        g1, g2 = _sc_gather(y1i, y2i, s.reshape(h_pad // _K, _K),
                            d.reshape(h_pad // _K, _K), h_pad)
        ea = lax.dynamic_slice_in_dim(edge_attr, lo, n)
        outs.append(_edge_stage(g1, g2, ea, W2, b2, gamma, beta, n))
    return jnp.concatenate(outs, axis=0)


# speedup vs baseline: 1.1946x; 1.1946x over previous
import functools

import jax
import jax.numpy as jnp
from jax import lax
from jax.experimental import pallas as pl
from jax.experimental.pallas import tpu as pltpu
from jax.experimental.pallas import tpu_sc as plsc


def _pack_bf16_halves(y):
    D = y.shape[1]
    a = lax.bitcast_convert_type(y[:, : D // 2], jnp.uint32)
    b = lax.bitcast_convert_type(y[:, D // 2:], jnp.uint32)
    ar = (a + 0x7FFF + ((a >> 16) & 1)) >> 16
    br = (b + 0x7FFF + ((b >> 16) & 1)) >> 16
    return lax.bitcast_convert_type((br << 16) | ar, jnp.int32)


def _proj_body(x_ref, w1a_ref, w1b_ref, b1_ref, y1_ref, y2_ref):
    xb = x_ref[...]
    y1 = (
        jnp.dot(xb, w1a_ref[...], preferred_element_type=jnp.float32)
        + b1_ref[...]
    )
    y2 = jnp.dot(xb, w1b_ref[...], preferred_element_type=jnp.float32)
    y1_ref[...] = _pack_bf16_halves(y1)
    y2_ref[...] = _pack_bf16_halves(y2)


def _node_proj(x, W1a, W1b, b1):
    N, D = x.shape
    NB = 1000
    return pl.pallas_call(
        _proj_body,
        grid=(N // NB,),
        in_specs=[
            pl.BlockSpec((NB, D), lambda i: (i, 0)),
            pl.BlockSpec((D, D), lambda i: (0, 0)),
            pl.BlockSpec((D, D), lambda i: (0, 0)),
            pl.BlockSpec((1, D), lambda i: (0, 0)),
        ],
        out_specs=[
            pl.BlockSpec((NB, D // 2), lambda i: (i, 0)),
            pl.BlockSpec((NB, D // 2), lambda i: (i, 0)),
        ],
        out_shape=[jax.ShapeDtypeStruct((N, D // 2), jnp.int32)] * 2,
    )(x, W1a, W1b, b1.reshape(1, D))


_NC = 2
_NS = 16
_NW = _NC * _NS
_K = 64
_NB = 4


def _sc_gather(y1i, y2i, src2, dst2, e_pad):
    W = y1i.shape[1]
    nch = e_pad // (_NW * _K)
    pw = nch * _K
    mesh = plsc.VectorSubcoreMesh(core_axis_name="c", subcore_axis_name="s")

    @functools.partial(
        pl.kernel,
        mesh=mesh,
        out_type=[jax.ShapeDtypeStruct((e_pad, W), jnp.int32)] * 2,
        scratch_types=[
            pltpu.VMEM((nch, _K), jnp.int32),
            pltpu.VMEM((nch, _K), jnp.int32),
            [pltpu.VMEM((_K, W), jnp.int32)] * _NB,
            [pltpu.VMEM((_K, W), jnp.int32)] * _NB,
            [pltpu.SemaphoreType.DMA] * _NB,
            [pltpu.SemaphoreType.DMA] * _NB,
        ],
    )
    def gather_kernel(y1_hbm, y2_hbm, src_hbm, dst_hbm, g1_hbm, g2_hbm,
                      isa, ida, r1, r2, sg, sw):
        wid = lax.axis_index("s") * _NC + lax.axis_index("c")
        base_row = wid * nch
        base = wid * pw

        pltpu.sync_copy(src_hbm.at[pl.ds(base_row, nch)], isa)
        pltpu.sync_copy(dst_hbm.at[pl.ds(base_row, nch)], ida)

        def issue_gathers(j, b):
            pltpu.async_copy(y1_hbm.at[isa.at[j]], r1[b], sg[b])
            pltpu.async_copy(y2_hbm.at[ida.at[j]], r2[b], sg[b])

        def wait_gathers(j, b):
            pltpu.make_async_copy(y1_hbm.at[isa.at[j]], r1[b], sg[b]).wait()
            pltpu.make_async_copy(y2_hbm.at[ida.at[j]], r2[b], sg[b]).wait()

        def o1(j):
            return g1_hbm.at[pl.ds(base + j * _K, _K)]

        def o2(j):
            return g2_hbm.at[pl.ds(base + j * _K, _K)]

        def issue_writes(j, b):
            pltpu.async_copy(r1[b], o1(j), sw[b])
            pltpu.async_copy(r2[b], o2(j), sw[b])

        def wait_writes(j, b):
            pltpu.make_async_copy(r1[b], o1(j), sw[b]).wait()
            pltpu.make_async_copy(r2[b], o2(j), sw[b]).wait()

        def sub(j, b, wait_w, issue_n):
            wait_gathers(j, b)
            issue_writes(j, b)
            if issue_n:
                bn = (b + 2) % _NB
                if wait_w:
                    wait_writes(j - 2, bn)
                issue_gathers(j + 2, bn)

        issue_gathers(0, 0)
        issue_gathers(1, 1)
        sub(0, 0, wait_w=False, issue_n=True)
        sub(1, 1, wait_w=False, issue_n=True)

        def step(q, carry):
            c0 = 4 * q + 2
            sub(c0, 2, wait_w=True, issue_n=True)
            sub(c0 + 1, 3, wait_w=True, issue_n=True)
            sub(c0 + 2, 0, wait_w=True, issue_n=True)
            sub(c0 + 3, 1, wait_w=True, issue_n=True)
            return carry

        lax.fori_loop(0, (nch - 4) // 4, step, 0)

        sub(nch - 2, 2, wait_w=False, issue_n=False)
        sub(nch - 1, 3, wait_w=False, issue_n=False)
        wait_writes(nch - 4, 0)
        wait_writes(nch - 3, 1)
        wait_writes(nch - 2, 2)
        wait_writes(nch - 1, 3)

    return gather_kernel(y1i, y2i, src2, dst2)


def _unpack_lo(u):
    return lax.bitcast_convert_type(u << 16, jnp.float32)


def _unpack_hi(u):
    return lax.bitcast_convert_type(u & jnp.uint32(0xFFFF0000), jnp.float32)


def _edge_body(g1_ref, g2_ref, ea_ref, w2a_ref, w2b_ref, b2_ref, gm_ref,
               bt_ref, o_ref):
    g1 = lax.bitcast_convert_type(g1_ref[...], jnp.uint32)
    g2 = lax.bitcast_convert_type(g2_ref[...], jnp.uint32)
    ta = _unpack_lo(g1) + _unpack_lo(g2)
    tb = _unpack_hi(g1) + _unpack_hi(g2)
    ha = jnp.where(ta > 0, ta, jnp.exp(ta) - 1.0)
    hb = jnp.where(tb > 0, tb, jnp.exp(tb) - 1.0)
    score = (
        jnp.dot(ha, w2a_ref[...], preferred_element_type=jnp.float32)
        + jnp.dot(hb, w2b_ref[...], preferred_element_type=jnp.float32)
        + b2_ref[...]
    )
    r = ea_ref[...] + score
    mu = jnp.mean(r, axis=1, keepdims=True)
    c = r - mu
    var = jnp.mean(c * c, axis=1, keepdims=True)
    o_ref[...] = c * lax.rsqrt(var + 1e-5) * gm_ref[...] + bt_ref[...]


def _edge_stage(g1, g2, edge_attr, W2, b2, gamma, beta, E):
    D = edge_attr.shape[1]
    EB = 1600
    return pl.pallas_call(
        _edge_body,
        grid=(E // EB,),
        in_specs=[
            pl.BlockSpec((EB, D // 2), lambda i: (i, 0)),
            pl.BlockSpec((EB, D // 2), lambda i: (i, 0)),
            pl.BlockSpec((EB, D), lambda i: (i, 0)),
            pl.BlockSpec((D // 2, D), lambda i: (0, 0)),
            pl.BlockSpec((D // 2, D), lambda i: (0, 0)),
            pl.BlockSpec((1, D), lambda i: (0, 0)),
            pl.BlockSpec((1, D), lambda i: (0, 0)),
            pl.BlockSpec((1, D), lambda i: (0, 0)),
        ],
        out_specs=pl.BlockSpec((EB, D), lambda i: (i, 0)),
        out_shape=jax.ShapeDtypeStruct((E, D), jnp.float32),
    )(g1, g2, edge_attr, W2[: D // 2], W2[D // 2:], b2.reshape(1, D),
      gamma.reshape(1, D), beta.reshape(1, D))


def kernel(x, edge_index, edge_attr, W1, b1, W2, b2, gamma, beta):
    N, D = x.shape
    E = edge_index.shape[1]

    W1a = W1[:D]
    W1b = W1[D:]
    y1i, y2i = _node_proj(x, W1a, W1b, b1)

    quantum = _NB * _NW * _K
    e_pad = ((E + quantum - 1) // quantum) * quantum
    src = edge_index[0].astype(jnp.int32)
    dst = edge_index[1].astype(jnp.int32)
    if e_pad != E:
        pad = e_pad - E
        src = jnp.concatenate([src, jnp.zeros((pad,), jnp.int32)])
        dst = jnp.concatenate([dst, jnp.zeros((pad,), jnp.int32)])
    src2 = src.reshape(e_pad // _K, _K)
    dst2 = dst.reshape(e_pad // _K, _K)

    g1, g2 = _sc_gather(y1i, y2i, src2, dst2, e_pad)

    return _edge_stage(g1, g2, edge_attr, W2, b2, gamma, beta, E)
